# Initial kernel scaffold; baseline (speedup 1.0000x reference)
import jax
import jax.numpy as jnp
from jax.experimental import pallas as pl


def kernel(x_real, x_imag, attn_real, attn_imag, alpha):
    raise NotImplementedError("write your pallas kernel here")


# speedup vs baseline: 3.1685x; 3.1685x over previous
import functools

import jax
import jax.numpy as jnp
from jax import lax
from jax.experimental import pallas as pl
from jax.experimental.pallas import tpu as pltpu
from jax.experimental.pallas import tpu_sc as plsc

B = 1024
F = 128
N_ATTN = 100000
N = B + N_ATTN
T = 1024
STEPS = 100
N_PAD = STEPS * T
G = 128
TBLK = T // G
NBLK = N_PAD // G
K = 32
NEG = -1e30


def _sims_body(xr_ref, xi_ref, ar_ref, ai_ref, sims_ref, bmax_ref, qn_ref):
    step = pl.program_id(0)

    @pl.when(step == 0)
    def _():
        q = jnp.concatenate([xr_ref[...], xi_ref[...]], axis=1)
        qss = jnp.sum(q * q, axis=1, keepdims=True)
        qn_ref[...] = q / jnp.maximum(jnp.sqrt(qss), 1e-12)

    dr = jnp.where(step == 0, xr_ref[...], ar_ref[...])
    di = jnp.where(step == 0, xi_ref[...], ai_ref[...])
    d = jnp.concatenate([dr, di], axis=1)
    dss = jnp.sum(d * d, axis=1, keepdims=True)
    dn = d / jnp.maximum(jnp.sqrt(dss), 1e-12)
    dims = (((1,), (1,)), ((), ()))
    sims = lax.dot_general(qn_ref[...], dn, dims,
                           preferred_element_type=jnp.float32)
    col = step * T + lax.broadcasted_iota(jnp.int32, (B, T), 1)
    sims = jnp.where(col < N, sims, NEG)
    sims_ref[...] = sims
    bmax_ref[0] = jnp.max(sims.reshape(B, TBLK, G), axis=2)


def _sims(x_real, x_imag, attn_real, attn_imag):
    return pl.pallas_call(
        _sims_body,
        grid=(STEPS,),
        in_specs=[
            pl.BlockSpec((B, F), lambda i: (0, 0)),
            pl.BlockSpec((B, F), lambda i: (0, 0)),
            pl.BlockSpec((T, F), lambda i: (jnp.clip(i - 1, 0, 97), 0)),
            pl.BlockSpec((T, F), lambda i: (jnp.clip(i - 1, 0, 97), 0)),
        ],
        out_specs=[
            pl.BlockSpec((B, T), lambda i: (0, i)),
            pl.BlockSpec((1, B, TBLK), lambda i: (i, 0, 0)),
        ],
        out_shape=[
            jax.ShapeDtypeStruct((B, N_PAD), jnp.float32),
            jax.ShapeDtypeStruct((STEPS, B, TBLK), jnp.float32),
        ],
        scratch_shapes=[
            pltpu.VMEM((B, 2 * F), jnp.float32),
        ],
        compiler_params=pltpu.CompilerParams(
            dimension_semantics=("arbitrary",),
        ),
    )(x_real, x_imag, attn_real, attn_imag)


_NC = 2
_NS = 16
_NW = _NC * _NS
_QPW = B // _NW
_NBV = NBLK // 16
_GV = G // 16

@functools.cache
def _sc_mesh():
    return plsc.VectorSubcoreMesh(
        core_axis_name="c", subcore_axis_name="s",
        num_cores=_NC, num_subcores=_NS)


def _topk_body(bmax_hbm, simsb_hbm, xr_hbm, xi_hbm, ar_hbm, ai_hbm,
               outr_hbm, outi_hbm,
               bvec, topblk, gblk, blocks, nidx, nax, nxr, nxi, nar, nai,
               sem):
    wid = lax.axis_index("s") * _NC + lax.axis_index("c")
    lane = lax.broadcasted_iota(jnp.int32, (16,), 0)
    lane0 = lane == 0
    ninf = jnp.full((16,), -jnp.inf, jnp.float32)

    def per_query(j, carry):
        q = wid * _QPW + j
        pltpu.sync_copy(bmax_hbm.at[q], bvec)

        def sel_blk(it, carry):
            def scan(i, mc):
                mv, mi = mc
                v = bvec[pl.ds(i * 16, 16)]
                idx = i * 16 + lane
                upd = v > mv
                return jnp.where(upd, v, mv), jnp.where(upd, idx, mi)

            mv, mi = lax.fori_loop(0, _NBV, scan,
                                   (ninf, jnp.zeros((16,), jnp.int32)))
            m = plsc.cummax(mv)[15]
            g = -plsc.cummax(-jnp.where(mv == m, mi, jnp.int32(1 << 30)))[15]
            plsc.store_scatter(bvec, [jnp.full((16,), g, jnp.int32)], ninf,
                               mask=lane0)
            plsc.store_scatter(topblk, [jnp.full((16,), it, jnp.int32)],
                               jnp.full((16,), g, jnp.int32), mask=lane0)
            return carry

        lax.fori_loop(0, K, sel_blk, 0)

        base = jnp.full((16,), q * NBLK, jnp.int32)
        gblk[pl.ds(0, 16)] = base + topblk[pl.ds(0, 16)]
        gblk[pl.ds(16, 16)] = base + topblk[pl.ds(16, 16)]
        pltpu.async_copy(simsb_hbm.at[gblk], blocks, sem).wait()

        def sel_elem(it, carry):
            def scan(i, mc):
                mv, ml = mc
                b = i // _GV
                l = i % _GV
                v = blocks[b, pl.ds(l * 16, 16)]
                lidx = i * 16 + lane
                upd = v > mv
                return jnp.where(upd, v, mv), jnp.where(upd, lidx, ml)

            mv, ml = lax.fori_loop(0, K * _GV, scan,
                                   (ninf, jnp.zeros((16,), jnp.int32)))
            m = plsc.cummax(mv)[15]
            loc = -plsc.cummax(-jnp.where(mv == m, ml, jnp.int32(1 << 30)))[15]
            b = loc // G
            off = loc % G
            gcol = topblk[pl.ds(b, 16)][0] * G + off
            plsc.store_scatter(blocks,
                               [jnp.full((16,), b, jnp.int32),
                                jnp.full((16,), off, jnp.int32)],
                               ninf, mask=lane0)
            plsc.store_scatter(nidx, [jnp.full((16,), it, jnp.int32)],
                               jnp.full((16,), gcol, jnp.int32), mask=lane0)
            return carry

        lax.fori_loop(0, K, sel_elem, 0)

        ix0 = jnp.minimum(nidx[pl.ds(0, 16)], B - 1)
        ix1 = jnp.minimum(nidx[pl.ds(16, 16)], B - 1)
        ia0 = jnp.clip(nidx[pl.ds(0, 16)] - B, 0, N_ATTN - 1)
        ia1 = jnp.clip(nidx[pl.ds(16, 16)] - B, 0, N_ATTN - 1)
        nax[pl.ds(0, 16)] = ix0
        nax[pl.ds(16, 16)] = ix1
        cx = pltpu.async_copy(xr_hbm.at[nax], nxr, sem)
        cx.wait()
        cx = pltpu.async_copy(xi_hbm.at[nax], nxi, sem)
        cx.wait()
        nax[pl.ds(0, 16)] = ia0
        nax[pl.ds(16, 16)] = ia1
        cx = pltpu.async_copy(ar_hbm.at[nax], nar, sem)
        cx.wait()
        cx = pltpu.async_copy(ai_hbm.at[nax], nai, sem)
        cx.wait()

        def sel_src(b, carry):
            is_x = nidx[pl.ds(b, 16)][0] < B
            for l in range(F // 16):
                sl = pl.ds(l * 16, 16)
                nxr[b, sl] = jnp.where(is_x, nxr[b, sl], nar[b, sl])
                nxi[b, sl] = jnp.where(is_x, nxi[b, sl], nai[b, sl])
            return carry

        lax.fori_loop(0, K, sel_src, 0)
        pltpu.sync_copy(nxr, outr_hbm.at[q])
        pltpu.sync_copy(nxi, outi_hbm.at[q])
        return carry

    lax.fori_loop(0, _QPW, per_query, 0)


def _topk_gather(bmax, simsb, x_real, x_imag, attn_real, attn_imag):
    f32 = jnp.float32
    i32 = jnp.int32
    return pl.kernel(
        _topk_body,
        out_type=[
            jax.ShapeDtypeStruct((B, K, F), f32),
            jax.ShapeDtypeStruct((B, K, F), f32),
        ],
        mesh=_sc_mesh(),
        compiler_params=pltpu.CompilerParams(needs_layout_passes=False),
        scratch_types=[
            pltpu.VMEM((NBLK,), f32),
            pltpu.VMEM((K + 16,), i32),
            pltpu.VMEM((K,), i32),
            pltpu.VMEM((K, G), f32),
            pltpu.VMEM((K + 16,), i32),
            pltpu.VMEM((K,), i32),
            pltpu.VMEM((K, F), f32),
            pltpu.VMEM((K, F), f32),
            pltpu.VMEM((K, F), f32),
            pltpu.VMEM((K, F), f32),
            pltpu.SemaphoreType.DMA,
        ],
    )(bmax, simsb, x_real, x_imag, attn_real, attn_imag)


QC = 256


def _combine_body(nr_ref, ni_ref, xr_ref, xi_ref, al_ref, or_ref, oi_ref):
    nr = nr_ref[...]
    ni = ni_ref[...]
    h = jnp.sqrt(nr * nr + ni * ni)
    hs = jnp.maximum(h, 1e-30)
    cosp = jnp.where(h > 0, nr / hs, 1.0)
    sinp = jnp.where(h > 0, ni / hs, 0.0)
    mean_rho = jnp.mean(h, axis=1) + 1e-7
    c = jnp.mean(cosp, axis=1)
    s = jnp.mean(sinp, axis=1)
    n = jnp.sqrt(c * c + s * s)
    ns = jnp.maximum(n, 1e-30)
    cosm = jnp.where(n > 0, c / ns, 1.0)
    sinm = jnp.where(n > 0, s / ns, 0.0)
    a = jnp.clip(al_ref[0], 0.0, 1.0)
    or_ref[...] = (1.0 - a) * xr_ref[...] + a * (mean_rho * cosm)
    oi_ref[...] = (1.0 - a) * xi_ref[...] + a * (mean_rho * sinm)


def _combine(nbr_r, nbr_i, x_real, x_imag, alpha):
    return pl.pallas_call(
        _combine_body,
        grid=(B // QC,),
        in_specs=[
            pl.BlockSpec((QC, K, F), lambda i: (i, 0, 0)),
            pl.BlockSpec((QC, K, F), lambda i: (i, 0, 0)),
            pl.BlockSpec((QC, F), lambda i: (i, 0)),
            pl.BlockSpec((QC, F), lambda i: (i, 0)),
            pl.BlockSpec(memory_space=pltpu.SMEM),
        ],
        out_specs=[
            pl.BlockSpec((QC, F), lambda i: (i, 0)),
            pl.BlockSpec((QC, F), lambda i: (i, 0)),
        ],
        out_shape=[
            jax.ShapeDtypeStruct((B, F), jnp.float32),
            jax.ShapeDtypeStruct((B, F), jnp.float32),
        ],
        compiler_params=pltpu.CompilerParams(
            dimension_semantics=("arbitrary",),
        ),
    )(nbr_r, nbr_i, x_real, x_imag, alpha)


@jax.jit
def kernel(x_real, x_imag, attn_real, attn_imag, alpha):
    sims, bmax3 = _sims(x_real, x_imag, attn_real, attn_imag)
    bmax = jnp.transpose(bmax3, (1, 0, 2)).reshape(B, NBLK)
    simsb = sims.reshape(B * NBLK, G)
    nbr_r, nbr_i = _topk_gather(bmax, simsb, x_real, x_imag,
                                attn_real, attn_imag)
    out_r, out_i = _combine(nbr_r, nbr_i, x_real, x_imag,
                            alpha.reshape(1))
    return jnp.stack([out_r, out_i], axis=-1)


# speedup vs baseline: 3.9584x; 1.2493x over previous
import functools

import jax
import jax.numpy as jnp
from jax import lax
from jax.experimental import pallas as pl
from jax.experimental.pallas import tpu as pltpu
from jax.experimental.pallas import tpu_sc as plsc

B = 1024
F = 128
N_ATTN = 100000
N = B + N_ATTN
T = 1024
STEPS = 100
N_PAD = STEPS * T
G = 128
TBLK = T // G
NBLK = N_PAD // G
K = 32
NEG = -1e30


def _sims_body(xr_ref, xi_ref, ar_ref, ai_ref, sims_ref, bmax_ref, qn_ref):
    step = pl.program_id(0)

    @pl.when(step == 0)
    def _():
        q = jnp.concatenate([xr_ref[...], xi_ref[...]], axis=1)
        qss = jnp.sum(q * q, axis=1, keepdims=True)
        qn_ref[...] = q / jnp.maximum(jnp.sqrt(qss), 1e-12)

    dr = jnp.where(step == 0, xr_ref[...], ar_ref[...])
    di = jnp.where(step == 0, xi_ref[...], ai_ref[...])
    d = jnp.concatenate([dr, di], axis=1)
    dss = jnp.sum(d * d, axis=1, keepdims=True)
    dn = d / jnp.maximum(jnp.sqrt(dss), 1e-12)
    dims = (((1,), (1,)), ((), ()))
    sims = lax.dot_general(qn_ref[...], dn, dims,
                           preferred_element_type=jnp.float32)
    col = step * T + lax.broadcasted_iota(jnp.int32, (B, T), 1)
    sims = jnp.where(col < N, sims, NEG)
    sims_ref[...] = sims
    bmax_ref[0] = jnp.max(sims.reshape(B, TBLK, G), axis=2)


def _sims(x_real, x_imag, attn_real, attn_imag):
    return pl.pallas_call(
        _sims_body,
        grid=(STEPS,),
        in_specs=[
            pl.BlockSpec((B, F), lambda i: (0, 0)),
            pl.BlockSpec((B, F), lambda i: (0, 0)),
            pl.BlockSpec((T, F), lambda i: (jnp.clip(i - 1, 0, 97), 0)),
            pl.BlockSpec((T, F), lambda i: (jnp.clip(i - 1, 0, 97), 0)),
        ],
        out_specs=[
            pl.BlockSpec((B, T), lambda i: (0, i)),
            pl.BlockSpec((1, B, TBLK), lambda i: (i, 0, 0)),
        ],
        out_shape=[
            jax.ShapeDtypeStruct((B, N_PAD), jnp.float32),
            jax.ShapeDtypeStruct((STEPS, B, TBLK), jnp.float32),
        ],
        scratch_shapes=[
            pltpu.VMEM((B, 2 * F), jnp.float32),
        ],
        compiler_params=pltpu.CompilerParams(
            dimension_semantics=("arbitrary",),
        ),
    )(x_real, x_imag, attn_real, attn_imag)


_NC = 2
_NS = 16
_NW = _NC * _NS
_QPW = B // _NW
_NBV = NBLK // 16
_GV = G // 16

@functools.cache
def _sc_mesh():
    return plsc.VectorSubcoreMesh(
        core_axis_name="c", subcore_axis_name="s",
        num_cores=_NC, num_subcores=_NS)


def _topk_body(bmax_hbm, simsb_hbm, xr_hbm, xi_hbm, ar_hbm, ai_hbm,
               outr_hbm, outi_hbm,
               bvec, cand_v, cand_i, topblk, gblk, blocks, nidx, nax,
               nxr, nxi, nar, nai, sem):
    wid = lax.axis_index("s") * _NC + lax.axis_index("c")
    lane = lax.broadcasted_iota(jnp.int32, (16,), 0)
    lane0 = lane == 0
    ninf = jnp.full((16,), -jnp.inf, jnp.float32)
    izero = jnp.zeros((16,), jnp.int32)
    big = jnp.int32(1 << 30)

    def select32(nv, get_id, record):
        def sel(it, carry):
            def scan(i, mc):
                mv, mi = mc
                v = cand_v[pl.ds(i * 16, 16)]
                upd = v > mv
                return jnp.where(upd, v, mv), jnp.where(upd, i * 16 + lane, mi)

            mv, mi = lax.fori_loop(0, nv, scan, (ninf, izero))
            m = plsc.cummax(mv)[15]
            p = -plsc.cummax(-jnp.where(mv == m, mi, big))[15]
            record(it, get_id(p))
            plsc.store_scatter(cand_v, [jnp.full((16,), p, jnp.int32)], ninf,
                               mask=lane0)
            return m

        return lax.fori_loop(0, K, sel, jnp.float32(0.0))

    def per_query(j, carry):
        q = wid * _QPW + j
        pltpu.sync_copy(bmax_hbm.at[q], bvec)

        ma = ninf
        mb = ninf
        for i in range(0, _NBV, 2):
            ma = jnp.maximum(ma, bvec[pl.ds(i * 16, 16)])
            mb = jnp.maximum(mb, bvec[pl.ds((i + 1) * 16, 16)])
        t1 = -plsc.cummax(-jnp.minimum(ma, mb))[15]
        tv1 = jnp.full((16,), t1, jnp.float32)

        cnt = jnp.int32(0)
        for i in range(_NBV):
            v = bvec[pl.ds(i * 16, 16)]
            msk = v >= tv1
            plsc.store_compressed(cand_v.at[pl.ds(cnt, 16)], v, mask=msk)
            plsc.store_compressed(cand_i.at[pl.ds(cnt, 16)],
                                  i * 16 + lane, mask=msk)
            cnt = cnt + plsc.all_reduce_population_count(msk)[0]
        cand_v[pl.ds(cnt, 16)] = ninf

        def rec_blk(it, blkid):
            plsc.store_scatter(topblk, [jnp.full((16,), it, jnp.int32)],
                               jnp.full((16,), blkid, jnp.int32), mask=lane0)

        tau1 = select32((cnt + 15) // 16,
                        lambda p: cand_i[pl.ds(p, 16)][0], rec_blk)

        base = jnp.full((16,), q * NBLK, jnp.int32)
        gblk[pl.ds(0, 16)] = base + topblk[pl.ds(0, 16)]
        gblk[pl.ds(16, 16)] = base + topblk[pl.ds(16, 16)]
        pltpu.async_copy(simsb_hbm.at[gblk], blocks, sem).wait()

        tv2 = jnp.full((16,), tau1, jnp.float32)

        def comp2(b, cnt):
            for l in range(_GV):
                v = blocks[b, pl.ds(l * 16, 16)]
                msk = v >= tv2
                plsc.store_compressed(cand_v.at[pl.ds(cnt, 16)], v, mask=msk)
                plsc.store_compressed(cand_i.at[pl.ds(cnt, 16)],
                                      b * G + l * 16 + lane, mask=msk)
                cnt = cnt + plsc.all_reduce_population_count(msk)[0]
            return cnt

        cnt = lax.fori_loop(0, K, comp2, jnp.int32(0))
        cand_v[pl.ds(cnt, 16)] = ninf

        def get_elem(p):
            loc = cand_i[pl.ds(p, 16)][0]
            b = loc // G
            off = loc % G
            return topblk[pl.ds(b, 16)][0] * G + off

        def rec_elem(it, gcol):
            plsc.store_scatter(nidx, [jnp.full((16,), it, jnp.int32)],
                               jnp.full((16,), gcol, jnp.int32), mask=lane0)

        select32((cnt + 15) // 16, get_elem, rec_elem)

        ix0 = jnp.minimum(nidx[pl.ds(0, 16)], B - 1)
        ix1 = jnp.minimum(nidx[pl.ds(16, 16)], B - 1)
        ia0 = jnp.clip(nidx[pl.ds(0, 16)] - B, 0, N_ATTN - 1)
        ia1 = jnp.clip(nidx[pl.ds(16, 16)] - B, 0, N_ATTN - 1)
        nax[pl.ds(0, 16)] = ix0
        nax[pl.ds(16, 16)] = ix1
        cx = pltpu.async_copy(xr_hbm.at[nax], nxr, sem)
        cx.wait()
        cx = pltpu.async_copy(xi_hbm.at[nax], nxi, sem)
        cx.wait()
        nax[pl.ds(0, 16)] = ia0
        nax[pl.ds(16, 16)] = ia1
        cx = pltpu.async_copy(ar_hbm.at[nax], nar, sem)
        cx.wait()
        cx = pltpu.async_copy(ai_hbm.at[nax], nai, sem)
        cx.wait()

        def sel_src(b, carry):
            is_x = nidx[pl.ds(b, 16)][0] < B
            for l in range(F // 16):
                sl = pl.ds(l * 16, 16)
                nxr[b, sl] = jnp.where(is_x, nxr[b, sl], nar[b, sl])
                nxi[b, sl] = jnp.where(is_x, nxi[b, sl], nai[b, sl])
            return carry

        lax.fori_loop(0, K, sel_src, 0)
        pltpu.sync_copy(nxr, outr_hbm.at[q])
        pltpu.sync_copy(nxi, outi_hbm.at[q])
        return carry

    lax.fori_loop(0, _QPW, per_query, 0)


def _topk_gather(bmax, simsb, x_real, x_imag, attn_real, attn_imag):
    f32 = jnp.float32
    i32 = jnp.int32
    return pl.kernel(
        _topk_body,
        out_type=[
            jax.ShapeDtypeStruct((B, K, F), f32),
            jax.ShapeDtypeStruct((B, K, F), f32),
        ],
        mesh=_sc_mesh(),
        compiler_params=pltpu.CompilerParams(needs_layout_passes=False),
        scratch_types=[
            pltpu.VMEM((NBLK,), f32),
            pltpu.VMEM((K * G + 32,), f32),
            pltpu.VMEM((K * G + 32,), i32),
            pltpu.VMEM((K + 16,), i32),
            pltpu.VMEM((K,), i32),
            pltpu.VMEM((K, G), f32),
            pltpu.VMEM((K + 16,), i32),
            pltpu.VMEM((K,), i32),
            pltpu.VMEM((K, F), f32),
            pltpu.VMEM((K, F), f32),
            pltpu.VMEM((K, F), f32),
            pltpu.VMEM((K, F), f32),
            pltpu.SemaphoreType.DMA,
        ],
    )(bmax, simsb, x_real, x_imag, attn_real, attn_imag)


QC = 256


def _combine_body(nr_ref, ni_ref, xr_ref, xi_ref, al_ref, or_ref, oi_ref):
    nr = nr_ref[...]
    ni = ni_ref[...]
    h = jnp.sqrt(nr * nr + ni * ni)
    hs = jnp.maximum(h, 1e-30)
    cosp = jnp.where(h > 0, nr / hs, 1.0)
    sinp = jnp.where(h > 0, ni / hs, 0.0)
    mean_rho = jnp.mean(h, axis=1) + 1e-7
    c = jnp.mean(cosp, axis=1)
    s = jnp.mean(sinp, axis=1)
    n = jnp.sqrt(c * c + s * s)
    ns = jnp.maximum(n, 1e-30)
    cosm = jnp.where(n > 0, c / ns, 1.0)
    sinm = jnp.where(n > 0, s / ns, 0.0)
    a = jnp.clip(al_ref[0], 0.0, 1.0)
    or_ref[...] = (1.0 - a) * xr_ref[...] + a * (mean_rho * cosm)
    oi_ref[...] = (1.0 - a) * xi_ref[...] + a * (mean_rho * sinm)


def _combine(nbr_r, nbr_i, x_real, x_imag, alpha):
    return pl.pallas_call(
        _combine_body,
        grid=(B // QC,),
        in_specs=[
            pl.BlockSpec((QC, K, F), lambda i: (i, 0, 0)),
            pl.BlockSpec((QC, K, F), lambda i: (i, 0, 0)),
            pl.BlockSpec((QC, F), lambda i: (i, 0)),
            pl.BlockSpec((QC, F), lambda i: (i, 0)),
            pl.BlockSpec(memory_space=pltpu.SMEM),
        ],
        out_specs=[
            pl.BlockSpec((QC, F), lambda i: (i, 0)),
            pl.BlockSpec((QC, F), lambda i: (i, 0)),
        ],
        out_shape=[
            jax.ShapeDtypeStruct((B, F), jnp.float32),
            jax.ShapeDtypeStruct((B, F), jnp.float32),
        ],
        compiler_params=pltpu.CompilerParams(
            dimension_semantics=("arbitrary",),
        ),
    )(nbr_r, nbr_i, x_real, x_imag, alpha)


@jax.jit
def kernel(x_real, x_imag, attn_real, attn_imag, alpha):
    sims, bmax3 = _sims(x_real, x_imag, attn_real, attn_imag)
    bmax = jnp.transpose(bmax3, (1, 0, 2)).reshape(B, NBLK)
    simsb = sims.reshape(B * NBLK, G)
    nbr_r, nbr_i = _topk_gather(bmax, simsb, x_real, x_imag,
                                attn_real, attn_imag)
    out_r, out_i = _combine(nbr_r, nbr_i, x_real, x_imag,
                            alpha.reshape(1))
    return jnp.stack([out_r, out_i], axis=-1)
